# baseline (speedup 1.0000x reference)
import functools

import jax
import jax.numpy as jnp
from jax import lax
from jax.experimental import pallas as pl
from jax.experimental.pallas import tpu as pltpu
from jax.experimental.pallas import tpu_sc as plsc

DIM = 128
_CHG = 256
_IPCG = _CHG // 128
_CHS = 128
_IPCS = _CHS // 128
_NW = 32

_MESH = dict(core_axis_name="c", subcore_axis_name="s")


@functools.lru_cache(maxsize=None)
def _make_gather(V, B, off_rows):
    rreal = B // 128
    rpw = (rreal + _NW - 1) // _NW
    rpw = (rpw + 7) // 8 * 8
    nk = rpw // _IPCG

    @functools.partial(
        pl.kernel,
        mesh=plsc.VectorSubcoreMesh(**_MESH),
        out_type=jax.ShapeDtypeStruct((B, DIM), jnp.float32),
        scratch_types=[
            pltpu.VMEM((rpw, 128), jnp.int32),
            pltpu.VMEM((2, _CHG, DIM), jnp.float32),
            pltpu.SemaphoreType.DMA,
            pltpu.SemaphoreType.DMA,
            pltpu.SemaphoreType.DMA,
        ],
    )
    def gk(table, idx, out, idx_all, rows_v, sem_g, sem_o0, sem_o1):
        w = lax.axis_index("c") * 16 + lax.axis_index("s")
        lrow0 = w * rpw
        pltpu.sync_copy(idx.at[pl.ds(off_rows + lrow0, rpw)], idx_all)
        nvalid = jnp.minimum(nk, (rreal - lrow0) // _IPCG)

        def chunk(k, buf, sem):
            hs = [
                pltpu.async_copy(
                    table.at[idx_all.at[_IPCG * k + j]],
                    rows_v.at[buf, pl.ds(j * 128, 128)],
                    sem_g,
                )
                for j in range(_IPCG)
            ]
            for h in hs:
                h.wait()
            pltpu.async_copy(
                rows_v.at[buf], out.at[pl.ds((lrow0 + _IPCG * k) * 128, _CHG)], sem
            )

        def drain(sem, buf):
            pltpu.make_async_copy(
                out.at[pl.ds(0, _CHG)], rows_v.at[buf], sem
            ).wait()

        def body(i, carry):
            k0 = 2 * i

            @pl.when(k0 < nvalid)
            def _():
                @pl.when(i > 0)
                def _():
                    drain(sem_o0, 0)
                chunk(k0, 0, sem_o0)

                @pl.when(k0 + 1 < nvalid)
                def _():
                    @pl.when(i > 0)
                    def _():
                        drain(sem_o1, 1)
                    chunk(k0 + 1, 1, sem_o1)

            return carry

        lax.fori_loop(0, (nk + 1) // 2, body, 0)

        @pl.when(nvalid > 0)
        def _():
            drain(sem_o0, 0)

        @pl.when(nvalid > 1)
        def _():
            drain(sem_o1, 1)

    return gk


@functools.lru_cache(maxsize=None)
def _make_scatter(V, B, off_rows):
    rreal = B // 128
    rpw = (rreal + _NW - 1) // _NW
    rpw = (rpw + 7) // 8 * 8
    nk = rpw // _IPCS
    rpt = (V // 16) // 8 * 8
    rem = V - 16 * rpt

    @functools.partial(
        pl.kernel,
        mesh=plsc.VectorSubcoreMesh(**_MESH),
        out_type=jax.ShapeDtypeStruct((2, V, DIM), jnp.float32),
        scratch_types=[
            pltpu.VMEM((rpw, 128), jnp.int32),
            pltpu.VMEM((2, _CHS, DIM), jnp.float32),
            pltpu.VMEM_SHARED((V, DIM), jnp.float32),
            pltpu.SemaphoreType.DMA,
            pltpu.SemaphoreType.DMA,
        ],
    )
    def sk(rows_hbm, idx_hbm, zeros_hbm, out, idx_all, rows_v, table,
           sem_l0, sem_l1):
        cid = lax.axis_index("c")
        sid = lax.axis_index("s")
        w = cid * 16 + sid
        lrow0 = w * rpw
        nvalid = jnp.minimum(nk, (rreal - lrow0) // _IPCS)
        sems = (sem_l0, sem_l1)

        def fire(k, buf):
            pltpu.async_copy(
                rows_hbm.at[pl.ds((lrow0 + _IPCS * k) * 128, _CHS)],
                rows_v.at[buf],
                sems[buf],
            )

        def drain(buf):
            pltpu.make_async_copy(
                rows_hbm.at[pl.ds(0, _CHS)], rows_v.at[buf], sems[buf]
            ).wait()

        pltpu.sync_copy(idx_hbm.at[pl.ds(off_rows + lrow0, rpw)], idx_all)

        @pl.when(nvalid > 0)
        def _():
            fire(0, 0)
        pltpu.sync_copy(
            zeros_hbm.at[pl.ds(sid * rpt, rpt)], table.at[pl.ds(sid * rpt, rpt)]
        )
        if rem:
            @pl.when(sid == 15)
            def _():
                pltpu.sync_copy(
                    zeros_hbm.at[pl.ds(16 * rpt, rem)],
                    table.at[pl.ds(16 * rpt, rem)],
                )
        plsc.subcore_barrier()

        def scat(k, buf):
            for j in range(_IPCS):
                pltpu.sync_copy(
                    rows_v.at[buf, pl.ds(j * 128, 128)],
                    table.at[idx_all.at[_IPCS * k + j]],
                    add=True,
                )

        def body(i, carry):
            k0 = 2 * i

            @pl.when(k0 < nvalid)
            def _():
                drain(0)

                @pl.when(k0 + 1 < nvalid)
                def _():
                    fire(k0 + 1, 1)
                scat(k0, 0)

                @pl.when(k0 + 1 < nvalid)
                def _():
                    drain(1)

                    @pl.when(k0 + 2 < nvalid)
                    def _():
                        fire(k0 + 2, 0)
                    scat(k0 + 1, 1)

            return carry

        lax.fori_loop(0, nk // 2, body, 0)
        plsc.subcore_barrier()
        pltpu.sync_copy(
            table.at[pl.ds(sid * rpt, rpt)], out.at[cid, pl.ds(sid * rpt, rpt)]
        )
        if rem:
            @pl.when(sid == 15)
            def _():
                pltpu.sync_copy(
                    table.at[pl.ds(16 * rpt, rem)],
                    out.at[cid, pl.ds(16 * rpt, rem)],
                )

    return sk


_BT = 3200


def _init_body(gx_ref, ea_ref, wx_ref, we_ref, inp_ref, msg_ref):
    acc = jnp.dot(gx_ref[...], wx_ref[...], preferred_element_type=jnp.float32)
    acc = acc + jnp.dot(ea_ref[...], we_ref[...], preferred_element_type=jnp.float32)
    inp_ref[...] = acc
    msg_ref[...] = jnp.maximum(acc, 0.0)


def _tc_init(gx, ea, wx, we, blk_off):
    e = gx.shape[0]
    nb = e // _BT
    return pl.pallas_call(
        _init_body,
        grid=(nb,),
        in_specs=[
            pl.BlockSpec((_BT, DIM), lambda i: (i, 0)),
            pl.BlockSpec((_BT, 16), lambda i, o=blk_off: (i + o, 0)),
            pl.BlockSpec((DIM, DIM), lambda i: (0, 0)),
            pl.BlockSpec((16, DIM), lambda i: (0, 0)),
        ],
        out_specs=[
            pl.BlockSpec((_BT, DIM), lambda i: (i, 0)),
            pl.BlockSpec((_BT, DIM), lambda i: (i, 0)),
        ],
        out_shape=[
            jax.ShapeDtypeStruct((e, DIM), jnp.float32),
            jax.ShapeDtypeStruct((e, DIM), jnp.float32),
        ],
    )(gx, ea, wx, we)


def _depth_body(msg_ref, g_ref, inp_ref, wh_ref, out_ref):
    msg = msg_ref[...]
    fwd = jnp.roll(msg, -1, axis=0)
    bwd = jnp.roll(msg, 1, axis=0)
    row = lax.broadcasted_iota(jnp.int32, msg.shape, 0)
    swapped = jnp.where((row & 1) == 0, fwd, bwd)
    t = g_ref[...] - swapped
    z = inp_ref[...] + jnp.dot(t, wh_ref[...], preferred_element_type=jnp.float32)
    out_ref[...] = jnp.maximum(z, 0.0)


def _tc_depth(msg, g, inp, wh_t):
    e = msg.shape[0]
    return pl.pallas_call(
        _depth_body,
        grid=(e // _BT,),
        in_specs=[
            pl.BlockSpec((_BT, DIM), lambda i: (i, 0)),
            pl.BlockSpec((_BT, DIM), lambda i: (i, 0)),
            pl.BlockSpec((_BT, DIM), lambda i: (i, 0)),
            pl.BlockSpec((DIM, DIM), lambda i: (0, 0)),
        ],
        out_specs=pl.BlockSpec((_BT, DIM), lambda i: (i, 0)),
        out_shape=jax.ShapeDtypeStruct((e, DIM), jnp.float32),
    )(msg, g, inp, wh_t)


def _combine_body(*refs):
    out_ref = refs[-1]
    acc = refs[0][0] + refs[0][1]
    for p_ref in refs[1:-1]:
        acc = acc + (p_ref[0] + p_ref[1])
    out_ref[...] = acc


def _tc_combine(parts):
    n = parts[0].shape[1]
    bn = 1000
    return pl.pallas_call(
        _combine_body,
        grid=(n // bn,),
        in_specs=[pl.BlockSpec((2, bn, DIM), lambda i: (0, i, 0))] * len(parts),
        out_specs=pl.BlockSpec((bn, DIM), lambda i: (i, 0)),
        out_shape=jax.ShapeDtypeStruct((n, DIM), jnp.float32),
    )(*parts)


def _final_body(x_ref, *refs):
    wox_ref, wos_ref, b_ref, out_ref = refs[-4:]
    s = refs[0][0] + refs[0][1]
    for p_ref in refs[1:-4]:
        s = s + (p_ref[0] + p_ref[1])
    z = jnp.dot(x_ref[...], wox_ref[...], preferred_element_type=jnp.float32)
    z = z + jnp.dot(s, wos_ref[...], preferred_element_type=jnp.float32)
    out_ref[...] = jnp.maximum(z + b_ref[...], 0.0)


def _tc_final(x, parts, wox, wos, b2):
    n = x.shape[0]
    bn = 1000
    return pl.pallas_call(
        _final_body,
        grid=(n // bn,),
        in_specs=[pl.BlockSpec((bn, DIM), lambda i: (i, 0))]
        + [pl.BlockSpec((2, bn, DIM), lambda i: (0, i, 0))] * len(parts)
        + [
            pl.BlockSpec((DIM, DIM), lambda i: (0, 0)),
            pl.BlockSpec((DIM, DIM), lambda i: (0, 0)),
            pl.BlockSpec((1, DIM), lambda i: (0, 0)),
        ],
        out_specs=pl.BlockSpec((bn, DIM), lambda i: (i, 0)),
        out_shape=jax.ShapeDtypeStruct((n, DIM), jnp.float32),
    )(x, *parts, wox, wos, b2)


_PIECES = (650, 1200, 650)


def _piece_layout(r):
    assert sum(_PIECES) == r
    out = []
    off = 0
    for rows in _PIECES:
        rpw = ((rows + _NW - 1) // _NW + 7) // 8 * 8
        out.append((off, rows, rpw * _NW))
        off += rpw * _NW
    return tuple(out)


def _prep_body(ei_ref, src_ref, dst_ref, dsts_ref):
    s = ei_ref[0]
    d = ei_ref[1]
    fwd = jnp.roll(d, -1, axis=1)
    bwd = jnp.roll(d, 1, axis=1)
    lane = lax.broadcasted_iota(jnp.int32, d.shape, 1)
    ds_ = jnp.where((lane & 1) == 0, fwd, bwd)

    layout = _piece_layout(s.shape[0])

    def padded(v):
        parts = []
        row = 0
        for _, rows, rpad in layout:
            parts.append(v[row:row + rows])
            parts.append(jnp.zeros((rpad - rows, 128), jnp.int32))
            row += rows
        return jnp.concatenate(parts, axis=0)

    src_ref[...] = padded(s)
    dst_ref[...] = padded(d)
    dsts_ref[...] = padded(ds_)


def _tc_prep(ei3, rpad_total):
    r = ei3.shape[1]
    return pl.pallas_call(
        _prep_body,
        grid=(1,),
        in_specs=[pl.BlockSpec((2, r, 128), lambda i: (0, 0, 0))],
        out_specs=[pl.BlockSpec((rpad_total, 128), lambda i: (0, 0))] * 3,
        out_shape=[jax.ShapeDtypeStruct((rpad_total, 128), jnp.int32)] * 3,
    )(ei3)


def kernel(x, edge_index, edge_attr, W_i, W_h, W_o, b_o):
    n = x.shape[0]
    e = edge_attr.shape[0]
    depth = 6

    r = e // 128
    layout = _piece_layout(r)
    rpad_total = layout[-1][0] + layout[-1][2]
    np_ = len(layout)
    ei3 = edge_index.astype(jnp.int32).reshape(2, r, 128)
    src_i, dst_i, dsts_i = _tc_prep(ei3, rpad_total)
    zeros_tab = jnp.zeros((n, DIM), jnp.float32)

    wx = W_i[:, :DIM].T
    we = W_i[:, DIM:].T
    wh_t = W_h.T
    wox = W_o[:, :DIM].T
    wos = W_o[:, DIM:].T
    b2 = b_o.reshape(1, DIM)

    gathers = [_make_gather(n, rows * 128, off) for off, rows, _ in layout]
    scatters = [_make_scatter(n, rows * 128, off) for off, rows, _ in layout]
    blk_offs = []
    row = 0
    for _, rows, _ in layout:
        blk_offs.append(row * 128 // _BT)
        row += rows

    gx = [gathers[p](x, src_i) for p in range(np_)]
    inp, msg = [None] * np_, [None] * np_
    for p in range(np_):
        inp[p], msg[p] = _tc_init(gx[p], edge_attr, wx, we, blk_offs[p])
    part = [scatters[p](msg[p], dst_i, zeros_tab) for p in range(np_)]
    for _ in range(depth - 1):
        esum = _tc_combine(part)
        g = [gathers[p](esum, dsts_i) for p in range(np_)]
        for p in range(np_):
            msg[p] = _tc_depth(msg[p], g[p], inp[p], wh_t)
            part[p] = scatters[p](msg[p], dst_i, zeros_tab)
    return _tc_final(x, part, wox, wos, b2)


# speedup vs baseline: 1.0328x; 1.0328x over previous
import functools

import jax
import jax.numpy as jnp
from jax import lax
from jax.experimental import pallas as pl
from jax.experimental.pallas import tpu as pltpu
from jax.experimental.pallas import tpu_sc as plsc

DIM = 128
_CHG = 256
_IPCG = _CHG // 128
_CHS = 128
_IPCS = _CHS // 128
_NW = 32

_MESH = dict(core_axis_name="c", subcore_axis_name="s")


@functools.lru_cache(maxsize=None)
def _make_gather(V, B, off_rows, dtype):
    rreal = B // 128
    rpw = (rreal + _NW - 1) // _NW
    rpw = (rpw + 7) // 8 * 8
    nk = rpw // _IPCG

    @functools.partial(
        pl.kernel,
        mesh=plsc.VectorSubcoreMesh(**_MESH),
        out_type=jax.ShapeDtypeStruct((B, DIM), dtype),
        scratch_types=[
            pltpu.VMEM((rpw, 128), jnp.int32),
            pltpu.VMEM((2, _CHG, DIM), dtype),
            pltpu.SemaphoreType.DMA,
            pltpu.SemaphoreType.DMA,
            pltpu.SemaphoreType.DMA,
        ],
    )
    def gk(table, idx, out, idx_all, rows_v, sem_g, sem_o0, sem_o1):
        w = lax.axis_index("c") * 16 + lax.axis_index("s")
        lrow0 = w * rpw
        pltpu.sync_copy(idx.at[pl.ds(off_rows + lrow0, rpw)], idx_all)
        nvalid = jnp.minimum(nk, (rreal - lrow0) // _IPCG)

        def chunk(k, buf, sem):
            hs = [
                pltpu.async_copy(
                    table.at[idx_all.at[_IPCG * k + j]],
                    rows_v.at[buf, pl.ds(j * 128, 128)],
                    sem_g,
                )
                for j in range(_IPCG)
            ]
            for h in hs:
                h.wait()
            pltpu.async_copy(
                rows_v.at[buf], out.at[pl.ds((lrow0 + _IPCG * k) * 128, _CHG)], sem
            )

        def drain(sem, buf):
            pltpu.make_async_copy(
                out.at[pl.ds(0, _CHG)], rows_v.at[buf], sem
            ).wait()

        def body(i, carry):
            k0 = 2 * i

            @pl.when(k0 < nvalid)
            def _():
                @pl.when(i > 0)
                def _():
                    drain(sem_o0, 0)
                chunk(k0, 0, sem_o0)

                @pl.when(k0 + 1 < nvalid)
                def _():
                    @pl.when(i > 0)
                    def _():
                        drain(sem_o1, 1)
                    chunk(k0 + 1, 1, sem_o1)

            return carry

        lax.fori_loop(0, (nk + 1) // 2, body, 0)

        @pl.when(nvalid > 0)
        def _():
            drain(sem_o0, 0)

        @pl.when(nvalid > 1)
        def _():
            drain(sem_o1, 1)

    return gk


@functools.lru_cache(maxsize=None)
def _make_scatter(V, B, off_rows):
    rreal = B // 128
    rpw = (rreal + _NW - 1) // _NW
    rpw = (rpw + 7) // 8 * 8
    nk = rpw // _IPCS
    rpt = (V // 16) // 8 * 8
    rem = V - 16 * rpt

    @functools.partial(
        pl.kernel,
        mesh=plsc.VectorSubcoreMesh(**_MESH),
        out_type=jax.ShapeDtypeStruct((2, V, DIM), jnp.float32),
        scratch_types=[
            pltpu.VMEM((rpw, 128), jnp.int32),
            pltpu.VMEM((2, _CHS, DIM), jnp.float32),
            pltpu.VMEM_SHARED((V, DIM), jnp.float32),
            pltpu.SemaphoreType.DMA,
            pltpu.SemaphoreType.DMA,
        ],
    )
    def sk(rows_hbm, idx_hbm, zeros_hbm, out, idx_all, rows_v, table,
           sem_l0, sem_l1):
        cid = lax.axis_index("c")
        sid = lax.axis_index("s")
        w = cid * 16 + sid
        lrow0 = w * rpw
        nvalid = jnp.minimum(nk, (rreal - lrow0) // _IPCS)
        sems = (sem_l0, sem_l1)

        def fire(k, buf):
            pltpu.async_copy(
                rows_hbm.at[pl.ds((lrow0 + _IPCS * k) * 128, _CHS)],
                rows_v.at[buf],
                sems[buf],
            )

        def drain(buf):
            pltpu.make_async_copy(
                rows_hbm.at[pl.ds(0, _CHS)], rows_v.at[buf], sems[buf]
            ).wait()

        pltpu.sync_copy(idx_hbm.at[pl.ds(off_rows + lrow0, rpw)], idx_all)

        @pl.when(nvalid > 0)
        def _():
            fire(0, 0)
        pltpu.sync_copy(
            zeros_hbm.at[pl.ds(sid * rpt, rpt)], table.at[pl.ds(sid * rpt, rpt)]
        )
        if rem:
            @pl.when(sid == 15)
            def _():
                pltpu.sync_copy(
                    zeros_hbm.at[pl.ds(16 * rpt, rem)],
                    table.at[pl.ds(16 * rpt, rem)],
                )
        plsc.subcore_barrier()

        def scat(k, buf):
            for j in range(_IPCS):
                pltpu.sync_copy(
                    rows_v.at[buf, pl.ds(j * 128, 128)],
                    table.at[idx_all.at[_IPCS * k + j]],
                    add=True,
                )

        def body(i, carry):
            k0 = 2 * i

            @pl.when(k0 < nvalid)
            def _():
                drain(0)

                @pl.when(k0 + 1 < nvalid)
                def _():
                    fire(k0 + 1, 1)
                scat(k0, 0)

                @pl.when(k0 + 1 < nvalid)
                def _():
                    drain(1)

                    @pl.when(k0 + 2 < nvalid)
                    def _():
                        fire(k0 + 2, 0)
                    scat(k0 + 1, 1)

            return carry

        lax.fori_loop(0, nk // 2, body, 0)
        plsc.subcore_barrier()
        pltpu.sync_copy(
            table.at[pl.ds(sid * rpt, rpt)], out.at[cid, pl.ds(sid * rpt, rpt)]
        )
        if rem:
            @pl.when(sid == 15)
            def _():
                pltpu.sync_copy(
                    table.at[pl.ds(16 * rpt, rem)],
                    out.at[cid, pl.ds(16 * rpt, rem)],
                )

    return sk


_BT = 3200


def _init_body(gx_ref, ea_ref, wx_ref, we_ref, inp_ref, msg_ref):
    acc = jnp.dot(gx_ref[...], wx_ref[...], preferred_element_type=jnp.float32)
    acc = acc + jnp.dot(ea_ref[...], we_ref[...], preferred_element_type=jnp.float32)
    inp_ref[...] = acc
    msg_ref[...] = jnp.maximum(acc, 0.0)


def _tc_init(gx, ea, wx, we, blk_off):
    e = gx.shape[0]
    nb = e // _BT
    return pl.pallas_call(
        _init_body,
        grid=(nb,),
        in_specs=[
            pl.BlockSpec((_BT, DIM), lambda i: (i, 0)),
            pl.BlockSpec((_BT, 16), lambda i, o=blk_off: (i + o, 0)),
            pl.BlockSpec((DIM, DIM), lambda i: (0, 0)),
            pl.BlockSpec((16, DIM), lambda i: (0, 0)),
        ],
        out_specs=[
            pl.BlockSpec((_BT, DIM), lambda i: (i, 0)),
            pl.BlockSpec((_BT, DIM), lambda i: (i, 0)),
        ],
        out_shape=[
            jax.ShapeDtypeStruct((e, DIM), jnp.float32),
            jax.ShapeDtypeStruct((e, DIM), jnp.float32),
        ],
    )(gx, ea, wx, we)


def _depth_body(msg_ref, g_ref, inp_ref, wh_ref, out_ref):
    msg = msg_ref[...]
    fwd = jnp.roll(msg, -1, axis=0)
    bwd = jnp.roll(msg, 1, axis=0)
    row = lax.broadcasted_iota(jnp.int32, msg.shape, 0)
    swapped = jnp.where((row & 1) == 0, fwd, bwd)
    t = g_ref[...].astype(jnp.float32) - swapped
    z = inp_ref[...] + jnp.dot(t, wh_ref[...], preferred_element_type=jnp.float32)
    out_ref[...] = jnp.maximum(z, 0.0)


def _tc_depth(msg, g, inp, wh_t):
    e = msg.shape[0]
    return pl.pallas_call(
        _depth_body,
        grid=(e // _BT,),
        in_specs=[
            pl.BlockSpec((_BT, DIM), lambda i: (i, 0)),
            pl.BlockSpec((_BT, DIM), lambda i: (i, 0)),
            pl.BlockSpec((_BT, DIM), lambda i: (i, 0)),
            pl.BlockSpec((DIM, DIM), lambda i: (0, 0)),
        ],
        out_specs=pl.BlockSpec((_BT, DIM), lambda i: (i, 0)),
        out_shape=jax.ShapeDtypeStruct((e, DIM), jnp.float32),
    )(msg, g, inp, wh_t)


def _combine_body(*refs):
    out_ref = refs[-1]
    acc = refs[0][0] + refs[0][1]
    for p_ref in refs[1:-1]:
        acc = acc + (p_ref[0] + p_ref[1])
    out_ref[...] = acc.astype(out_ref.dtype)


def _tc_combine(parts):
    n = parts[0].shape[1]
    bn = 1000
    return pl.pallas_call(
        _combine_body,
        grid=(n // bn,),
        in_specs=[pl.BlockSpec((2, bn, DIM), lambda i: (0, i, 0))] * len(parts),
        out_specs=pl.BlockSpec((bn, DIM), lambda i: (i, 0)),
        out_shape=jax.ShapeDtypeStruct((n, DIM), jnp.float32),
    )(*parts)


def _final_body(x_ref, *refs):
    wox_ref, wos_ref, b_ref, out_ref = refs[-4:]
    s = refs[0][0] + refs[0][1]
    for p_ref in refs[1:-4]:
        s = s + (p_ref[0] + p_ref[1])
    z = jnp.dot(x_ref[...], wox_ref[...], preferred_element_type=jnp.float32)
    z = z + jnp.dot(s, wos_ref[...], preferred_element_type=jnp.float32)
    out_ref[...] = jnp.maximum(z + b_ref[...], 0.0)


def _tc_final(x, parts, wox, wos, b2):
    n = x.shape[0]
    bn = 1000
    return pl.pallas_call(
        _final_body,
        grid=(n // bn,),
        in_specs=[pl.BlockSpec((bn, DIM), lambda i: (i, 0))]
        + [pl.BlockSpec((2, bn, DIM), lambda i: (0, i, 0))] * len(parts)
        + [
            pl.BlockSpec((DIM, DIM), lambda i: (0, 0)),
            pl.BlockSpec((DIM, DIM), lambda i: (0, 0)),
            pl.BlockSpec((1, DIM), lambda i: (0, 0)),
        ],
        out_specs=pl.BlockSpec((bn, DIM), lambda i: (i, 0)),
        out_shape=jax.ShapeDtypeStruct((n, DIM), jnp.float32),
    )(x, *parts, wox, wos, b2)


_PIECES = (1250, 1250)


def _piece_layout(r):
    assert sum(_PIECES) == r
    out = []
    off = 0
    for rows in _PIECES:
        rpw = ((rows + _NW - 1) // _NW + 7) // 8 * 8
        out.append((off, rows, rpw * _NW))
        off += rpw * _NW
    return tuple(out)


def _prep_body(ei_ref, src_ref, dst_ref, dsts_ref):
    s = ei_ref[0]
    d = ei_ref[1]
    fwd = jnp.roll(d, -1, axis=1)
    bwd = jnp.roll(d, 1, axis=1)
    lane = lax.broadcasted_iota(jnp.int32, d.shape, 1)
    ds_ = jnp.where((lane & 1) == 0, fwd, bwd)

    layout = _piece_layout(s.shape[0])

    def padded(v):
        parts = []
        row = 0
        for _, rows, rpad in layout:
            parts.append(v[row:row + rows])
            parts.append(jnp.zeros((rpad - rows, 128), jnp.int32))
            row += rows
        return jnp.concatenate(parts, axis=0)

    src_ref[...] = padded(s)
    dst_ref[...] = padded(d)
    dsts_ref[...] = padded(ds_)


def _tc_prep(ei3, rpad_total):
    r = ei3.shape[1]
    return pl.pallas_call(
        _prep_body,
        grid=(1,),
        in_specs=[pl.BlockSpec((2, r, 128), lambda i: (0, 0, 0))],
        out_specs=[pl.BlockSpec((rpad_total, 128), lambda i: (0, 0))] * 3,
        out_shape=[jax.ShapeDtypeStruct((rpad_total, 128), jnp.int32)] * 3,
    )(ei3)


def kernel(x, edge_index, edge_attr, W_i, W_h, W_o, b_o):
    n = x.shape[0]
    e = edge_attr.shape[0]
    depth = 6

    r = e // 128
    layout = _piece_layout(r)
    rpad_total = layout[-1][0] + layout[-1][2]
    np_ = len(layout)
    ei3 = edge_index.astype(jnp.int32).reshape(2, r, 128)
    src_i, dst_i, dsts_i = _tc_prep(ei3, rpad_total)
    zeros_tab = jnp.zeros((n, DIM), jnp.float32)

    wx = W_i[:, :DIM].T
    we = W_i[:, DIM:].T
    wh_t = W_h.T
    wox = W_o[:, :DIM].T
    wos = W_o[:, DIM:].T
    b2 = b_o.reshape(1, DIM)

    gathers_f = [_make_gather(n, rows * 128, off, jnp.float32) for off, rows, _ in layout]
    gathers_h = gathers_f
    scatters = [_make_scatter(n, rows * 128, off) for off, rows, _ in layout]
    blk_offs = []
    row = 0
    for _, rows, _ in layout:
        blk_offs.append(row * 128 // _BT)
        row += rows

    gx = [gathers_f[p](x, src_i) for p in range(np_)]
    inp, msg = [None] * np_, [None] * np_
    for p in range(np_):
        inp[p], msg[p] = _tc_init(gx[p], edge_attr, wx, we, blk_offs[p])
    part = [scatters[p](msg[p], dst_i, zeros_tab) for p in range(np_)]
    for _ in range(depth - 1):
        esum = _tc_combine(part)
        g = [gathers_h[p](esum, dsts_i) for p in range(np_)]
        for p in range(np_):
            msg[p] = _tc_depth(msg[p], g[p], inp[p], wh_t)
            part[p] = scatters[p](msg[p], dst_i, zeros_tab)
    return _tc_final(x, part, wox, wos, b2)


# speedup vs baseline: 1.0382x; 1.0053x over previous
import functools

import jax
import jax.numpy as jnp
from jax import lax
from jax.experimental import pallas as pl
from jax.experimental.pallas import tpu as pltpu
from jax.experimental.pallas import tpu_sc as plsc

DIM = 128
_CHG = 256
_IPCG = _CHG // 128
_CHS = 128
_IPCS = _CHS // 128
_NW = 32

_MESH = dict(core_axis_name="c", subcore_axis_name="s")


@functools.lru_cache(maxsize=None)
def _make_gather(V, B, off_rows, dtype):
    rreal = B // 128
    rpw = (rreal + _NW - 1) // _NW
    rpw = (rpw + 7) // 8 * 8
    nk = rpw // _IPCG

    @functools.partial(
        pl.kernel,
        mesh=plsc.VectorSubcoreMesh(**_MESH),
        out_type=jax.ShapeDtypeStruct((B, DIM), dtype),
        scratch_types=[
            pltpu.VMEM((rpw, 128), jnp.int32),
            pltpu.VMEM((2, _CHG, DIM), dtype),
            pltpu.SemaphoreType.DMA,
            pltpu.SemaphoreType.DMA,
            pltpu.SemaphoreType.DMA,
        ],
    )
    def gk(table, idx, out, idx_all, rows_v, sem_g, sem_o0, sem_o1):
        w = lax.axis_index("c") * 16 + lax.axis_index("s")
        lrow0 = w * rpw
        pltpu.sync_copy(idx.at[pl.ds(off_rows + lrow0, rpw)], idx_all)
        nvalid = jnp.minimum(nk, (rreal - lrow0) // _IPCG)

        def chunk(k, buf, sem):
            hs = [
                pltpu.async_copy(
                    table.at[idx_all.at[_IPCG * k + j]],
                    rows_v.at[buf, pl.ds(j * 128, 128)],
                    sem_g,
                )
                for j in range(_IPCG)
            ]
            for h in hs:
                h.wait()
            pltpu.async_copy(
                rows_v.at[buf], out.at[pl.ds((lrow0 + _IPCG * k) * 128, _CHG)], sem
            )

        def drain(sem, buf):
            pltpu.make_async_copy(
                out.at[pl.ds(0, _CHG)], rows_v.at[buf], sem
            ).wait()

        def body(i, carry):
            k0 = 2 * i

            @pl.when(k0 < nvalid)
            def _():
                @pl.when(i > 0)
                def _():
                    drain(sem_o0, 0)
                chunk(k0, 0, sem_o0)

                @pl.when(k0 + 1 < nvalid)
                def _():
                    @pl.when(i > 0)
                    def _():
                        drain(sem_o1, 1)
                    chunk(k0 + 1, 1, sem_o1)

            return carry

        lax.fori_loop(0, (nk + 1) // 2, body, 0)

        @pl.when(nvalid > 0)
        def _():
            drain(sem_o0, 0)

        @pl.when(nvalid > 1)
        def _():
            drain(sem_o1, 1)

    return gk


@functools.lru_cache(maxsize=None)
def _make_scatter(V, B, off_rows):
    rreal = B // 128
    rpw = (rreal + _NW - 1) // _NW
    rpw = (rpw + 7) // 8 * 8
    nk = rpw // _IPCS
    rpt = (V // 16) // 8 * 8
    rem = V - 16 * rpt

    @functools.partial(
        pl.kernel,
        mesh=plsc.VectorSubcoreMesh(**_MESH),
        out_type=jax.ShapeDtypeStruct((2, V, DIM), jnp.float32),
        scratch_types=[
            pltpu.VMEM((rpw, 128), jnp.int32),
            pltpu.VMEM((2, _CHS, DIM), jnp.float32),
            pltpu.VMEM_SHARED((V, DIM), jnp.float32),
            pltpu.SemaphoreType.DMA,
            pltpu.SemaphoreType.DMA,
        ],
    )
    def sk(rows_hbm, idx_hbm, zeros_hbm, out, idx_all, rows_v, table,
           sem_l0, sem_l1):
        cid = lax.axis_index("c")
        sid = lax.axis_index("s")
        w = cid * 16 + sid
        lrow0 = w * rpw
        nvalid = jnp.minimum(nk, (rreal - lrow0) // _IPCS)
        sems = (sem_l0, sem_l1)

        def fire(k, buf):
            pltpu.async_copy(
                rows_hbm.at[pl.ds((lrow0 + _IPCS * k) * 128, _CHS)],
                rows_v.at[buf],
                sems[buf],
            )

        def drain(buf):
            pltpu.make_async_copy(
                rows_hbm.at[pl.ds(0, _CHS)], rows_v.at[buf], sems[buf]
            ).wait()

        pltpu.sync_copy(idx_hbm.at[pl.ds(off_rows + lrow0, rpw)], idx_all)

        @pl.when(nvalid > 0)
        def _():
            fire(0, 0)
        pltpu.sync_copy(
            zeros_hbm.at[pl.ds(sid * rpt, rpt)], table.at[pl.ds(sid * rpt, rpt)]
        )
        if rem:
            @pl.when(sid == 15)
            def _():
                pltpu.sync_copy(
                    zeros_hbm.at[pl.ds(16 * rpt, rem)],
                    table.at[pl.ds(16 * rpt, rem)],
                )
        plsc.subcore_barrier()

        def scat(k, buf):
            for j in range(_IPCS):
                pltpu.sync_copy(
                    rows_v.at[buf, pl.ds(j * 128, 128)],
                    table.at[idx_all.at[_IPCS * k + j]],
                    add=True,
                )

        def body(i, carry):
            k0 = 2 * i

            @pl.when(k0 < nvalid)
            def _():
                drain(0)

                @pl.when(k0 + 1 < nvalid)
                def _():
                    fire(k0 + 1, 1)
                scat(k0, 0)

                @pl.when(k0 + 1 < nvalid)
                def _():
                    drain(1)

                    @pl.when(k0 + 2 < nvalid)
                    def _():
                        fire(k0 + 2, 0)
                    scat(k0 + 1, 1)

            return carry

        lax.fori_loop(0, nk // 2, body, 0)
        plsc.subcore_barrier()
        pltpu.sync_copy(
            table.at[pl.ds(sid * rpt, rpt)], out.at[cid, pl.ds(sid * rpt, rpt)]
        )
        if rem:
            @pl.when(sid == 15)
            def _():
                pltpu.sync_copy(
                    table.at[pl.ds(16 * rpt, rem)],
                    out.at[cid, pl.ds(16 * rpt, rem)],
                )

    return sk


_BT = 6400


def _init_body(gx_ref, ea_ref, wx_ref, we_ref, inp_ref, msg_ref):
    acc = jnp.dot(gx_ref[...], wx_ref[...], preferred_element_type=jnp.float32)
    acc = acc + jnp.dot(ea_ref[...], we_ref[...], preferred_element_type=jnp.float32)
    inp_ref[...] = acc
    msg_ref[...] = jnp.maximum(acc, 0.0)


def _tc_init(gx, ea, wx, we, blk_off):
    e = gx.shape[0]
    nb = e // _BT
    return pl.pallas_call(
        _init_body,
        grid=(nb,),
        in_specs=[
            pl.BlockSpec((_BT, DIM), lambda i: (i, 0)),
            pl.BlockSpec((_BT, 16), lambda i, o=blk_off: (i + o, 0)),
            pl.BlockSpec((DIM, DIM), lambda i: (0, 0)),
            pl.BlockSpec((16, DIM), lambda i: (0, 0)),
        ],
        out_specs=[
            pl.BlockSpec((_BT, DIM), lambda i: (i, 0)),
            pl.BlockSpec((_BT, DIM), lambda i: (i, 0)),
        ],
        out_shape=[
            jax.ShapeDtypeStruct((e, DIM), jnp.float32),
            jax.ShapeDtypeStruct((e, DIM), jnp.float32),
        ],
    )(gx, ea, wx, we)


def _depth_body(msg_ref, g_ref, inp_ref, wh_ref, out_ref):
    msg = msg_ref[...]
    fwd = jnp.roll(msg, -1, axis=0)
    bwd = jnp.roll(msg, 1, axis=0)
    row = lax.broadcasted_iota(jnp.int32, msg.shape, 0)
    swapped = jnp.where((row & 1) == 0, fwd, bwd)
    t = g_ref[...].astype(jnp.float32) - swapped
    z = inp_ref[...] + jnp.dot(t, wh_ref[...], preferred_element_type=jnp.float32)
    out_ref[...] = jnp.maximum(z, 0.0)


def _tc_depth(msg, g, inp, wh_t):
    e = msg.shape[0]
    return pl.pallas_call(
        _depth_body,
        grid=(e // _BT,),
        in_specs=[
            pl.BlockSpec((_BT, DIM), lambda i: (i, 0)),
            pl.BlockSpec((_BT, DIM), lambda i: (i, 0)),
            pl.BlockSpec((_BT, DIM), lambda i: (i, 0)),
            pl.BlockSpec((DIM, DIM), lambda i: (0, 0)),
        ],
        out_specs=pl.BlockSpec((_BT, DIM), lambda i: (i, 0)),
        out_shape=jax.ShapeDtypeStruct((e, DIM), jnp.float32),
    )(msg, g, inp, wh_t)


def _combine_body(*refs):
    out_ref = refs[-1]
    acc = refs[0][0] + refs[0][1]
    for p_ref in refs[1:-1]:
        acc = acc + (p_ref[0] + p_ref[1])
    out_ref[...] = acc.astype(out_ref.dtype)


def _tc_combine(parts):
    n = parts[0].shape[1]
    bn = 1000
    return pl.pallas_call(
        _combine_body,
        grid=(n // bn,),
        in_specs=[pl.BlockSpec((2, bn, DIM), lambda i: (0, i, 0))] * len(parts),
        out_specs=pl.BlockSpec((bn, DIM), lambda i: (i, 0)),
        out_shape=jax.ShapeDtypeStruct((n, DIM), jnp.float32),
    )(*parts)


def _final_body(x_ref, *refs):
    wox_ref, wos_ref, b_ref, out_ref = refs[-4:]
    s = refs[0][0] + refs[0][1]
    for p_ref in refs[1:-4]:
        s = s + (p_ref[0] + p_ref[1])
    z = jnp.dot(x_ref[...], wox_ref[...], preferred_element_type=jnp.float32)
    z = z + jnp.dot(s, wos_ref[...], preferred_element_type=jnp.float32)
    out_ref[...] = jnp.maximum(z + b_ref[...], 0.0)


def _tc_final(x, parts, wox, wos, b2):
    n = x.shape[0]
    bn = 1000
    return pl.pallas_call(
        _final_body,
        grid=(n // bn,),
        in_specs=[pl.BlockSpec((bn, DIM), lambda i: (i, 0))]
        + [pl.BlockSpec((2, bn, DIM), lambda i: (0, i, 0))] * len(parts)
        + [
            pl.BlockSpec((DIM, DIM), lambda i: (0, 0)),
            pl.BlockSpec((DIM, DIM), lambda i: (0, 0)),
            pl.BlockSpec((1, DIM), lambda i: (0, 0)),
        ],
        out_specs=pl.BlockSpec((bn, DIM), lambda i: (i, 0)),
        out_shape=jax.ShapeDtypeStruct((n, DIM), jnp.float32),
    )(x, *parts, wox, wos, b2)


_PIECES = (1250, 1250)


def _piece_layout(r):
    assert sum(_PIECES) == r
    out = []
    off = 0
    for rows in _PIECES:
        rpw = ((rows + _NW - 1) // _NW + 7) // 8 * 8
        out.append((off, rows, rpw * _NW))
        off += rpw * _NW
    return tuple(out)


def _prep_body(ei_ref, src_ref, dst_ref, dsts_ref):
    s = ei_ref[0]
    d = ei_ref[1]
    fwd = jnp.roll(d, -1, axis=1)
    bwd = jnp.roll(d, 1, axis=1)
    lane = lax.broadcasted_iota(jnp.int32, d.shape, 1)
    ds_ = jnp.where((lane & 1) == 0, fwd, bwd)

    layout = _piece_layout(s.shape[0])

    def padded(v):
        parts = []
        row = 0
        for _, rows, rpad in layout:
            parts.append(v[row:row + rows])
            parts.append(jnp.zeros((rpad - rows, 128), jnp.int32))
            row += rows
        return jnp.concatenate(parts, axis=0)

    src_ref[...] = padded(s)
    dst_ref[...] = padded(d)
    dsts_ref[...] = padded(ds_)


def _tc_prep(ei3, rpad_total):
    r = ei3.shape[1]
    return pl.pallas_call(
        _prep_body,
        grid=(1,),
        in_specs=[pl.BlockSpec((2, r, 128), lambda i: (0, 0, 0))],
        out_specs=[pl.BlockSpec((rpad_total, 128), lambda i: (0, 0))] * 3,
        out_shape=[jax.ShapeDtypeStruct((rpad_total, 128), jnp.int32)] * 3,
    )(ei3)


def kernel(x, edge_index, edge_attr, W_i, W_h, W_o, b_o):
    n = x.shape[0]
    e = edge_attr.shape[0]
    depth = 6

    r = e // 128
    layout = _piece_layout(r)
    rpad_total = layout[-1][0] + layout[-1][2]
    np_ = len(layout)
    ei3 = edge_index.astype(jnp.int32).reshape(2, r, 128)
    src_i, dst_i, dsts_i = _tc_prep(ei3, rpad_total)
    zeros_tab = jnp.zeros((n, DIM), jnp.float32)

    wx = W_i[:, :DIM].T
    we = W_i[:, DIM:].T
    wh_t = W_h.T
    wox = W_o[:, :DIM].T
    wos = W_o[:, DIM:].T
    b2 = b_o.reshape(1, DIM)

    gathers_f = [_make_gather(n, rows * 128, off, jnp.float32) for off, rows, _ in layout]
    gathers_h = gathers_f
    scatters = [_make_scatter(n, rows * 128, off) for off, rows, _ in layout]
    blk_offs = []
    row = 0
    for _, rows, _ in layout:
        blk_offs.append(row * 128 // _BT)
        row += rows

    gx = [gathers_f[p](x, src_i) for p in range(np_)]
    inp, msg = [None] * np_, [None] * np_
    for p in range(np_):
        inp[p], msg[p] = _tc_init(gx[p], edge_attr, wx, we, blk_offs[p])
    part = [scatters[p](msg[p], dst_i, zeros_tab) for p in range(np_)]
    for _ in range(depth - 1):
        esum = _tc_combine(part)
        g = [gathers_h[p](esum, dsts_i) for p in range(np_)]
        for p in range(np_):
            msg[p] = _tc_depth(msg[p], g[p], inp[p], wh_t)
            part[p] = scatters[p](msg[p], dst_i, zeros_tab)
    return _tc_final(x, part, wox, wos, b2)


# speedup vs baseline: 1.0385x; 1.0003x over previous
import functools

import jax
import jax.numpy as jnp
from jax import lax
from jax.experimental import pallas as pl
from jax.experimental.pallas import tpu as pltpu
from jax.experimental.pallas import tpu_sc as plsc

DIM = 128
_CHG = 256
_IPCG = _CHG // 128
_CHS = 128
_IPCS = _CHS // 128
_NW = 32

_MESH = dict(core_axis_name="c", subcore_axis_name="s")


@functools.lru_cache(maxsize=None)
def _make_gather(V, B, off_rows, dtype):
    rreal = B // 128
    rpw = (rreal + _NW - 1) // _NW
    rpw = (rpw + 7) // 8 * 8
    nk = rpw // _IPCG

    @functools.partial(
        pl.kernel,
        mesh=plsc.VectorSubcoreMesh(**_MESH),
        out_type=jax.ShapeDtypeStruct((B, DIM), dtype),
        scratch_types=[
            pltpu.VMEM((rpw, 128), jnp.int32),
            pltpu.VMEM((2, _CHG, DIM), dtype),
            pltpu.SemaphoreType.DMA,
            pltpu.SemaphoreType.DMA,
            pltpu.SemaphoreType.DMA,
        ],
    )
    def gk(table, idx, out, idx_all, rows_v, sem_g, sem_o0, sem_o1):
        w = lax.axis_index("c") * 16 + lax.axis_index("s")
        lrow0 = w * rpw
        pltpu.sync_copy(idx.at[pl.ds(off_rows + lrow0, rpw)], idx_all)
        nvalid = jnp.minimum(nk, (rreal - lrow0) // _IPCG)

        def chunk(k, buf, sem):
            hs = [
                pltpu.async_copy(
                    table.at[idx_all.at[_IPCG * k + j]],
                    rows_v.at[buf, pl.ds(j * 128, 128)],
                    sem_g,
                )
                for j in range(_IPCG)
            ]
            for h in hs:
                h.wait()
            pltpu.async_copy(
                rows_v.at[buf], out.at[pl.ds((lrow0 + _IPCG * k) * 128, _CHG)], sem
            )

        def drain(sem, buf):
            pltpu.make_async_copy(
                out.at[pl.ds(0, _CHG)], rows_v.at[buf], sem
            ).wait()

        def body(i, carry):
            k0 = 2 * i

            @pl.when(k0 < nvalid)
            def _():
                @pl.when(i > 0)
                def _():
                    drain(sem_o0, 0)
                chunk(k0, 0, sem_o0)

                @pl.when(k0 + 1 < nvalid)
                def _():
                    @pl.when(i > 0)
                    def _():
                        drain(sem_o1, 1)
                    chunk(k0 + 1, 1, sem_o1)

            return carry

        lax.fori_loop(0, (nk + 1) // 2, body, 0)

        @pl.when(nvalid > 0)
        def _():
            drain(sem_o0, 0)

        @pl.when(nvalid > 1)
        def _():
            drain(sem_o1, 1)

    return gk


@functools.lru_cache(maxsize=None)
def _make_scatter(V, B, off_rows):
    rreal = B // 128
    rpw = (rreal + _NW - 1) // _NW
    rpw = (rpw + 7) // 8 * 8
    nk = rpw // _IPCS
    rpt = (V // 16) // 8 * 8
    rem = V - 16 * rpt

    @functools.partial(
        pl.kernel,
        mesh=plsc.VectorSubcoreMesh(**_MESH),
        out_type=jax.ShapeDtypeStruct((2, V, DIM), jnp.float32),
        scratch_types=[
            pltpu.VMEM((rpw, 128), jnp.int32),
            pltpu.VMEM((2, _CHS, DIM), jnp.float32),
            pltpu.VMEM_SHARED((V, DIM), jnp.float32),
            pltpu.SemaphoreType.DMA,
            pltpu.SemaphoreType.DMA,
        ],
    )
    def sk(rows_hbm, idx_hbm, zeros_hbm, out, idx_all, rows_v, table,
           sem_l0, sem_l1):
        cid = lax.axis_index("c")
        sid = lax.axis_index("s")
        w = cid * 16 + sid
        lrow0 = w * rpw
        nvalid = jnp.minimum(nk, (rreal - lrow0) // _IPCS)
        sems = (sem_l0, sem_l1)

        def fire(k, buf):
            pltpu.async_copy(
                rows_hbm.at[pl.ds((lrow0 + _IPCS * k) * 128, _CHS)],
                rows_v.at[buf],
                sems[buf],
            )

        def drain(buf):
            pltpu.make_async_copy(
                rows_hbm.at[pl.ds(0, _CHS)], rows_v.at[buf], sems[buf]
            ).wait()

        pltpu.sync_copy(idx_hbm.at[pl.ds(off_rows + lrow0, rpw)], idx_all)

        @pl.when(nvalid > 0)
        def _():
            fire(0, 0)
        pltpu.sync_copy(
            zeros_hbm.at[pl.ds(sid * rpt, rpt)], table.at[pl.ds(sid * rpt, rpt)]
        )
        if rem:
            @pl.when(sid == 15)
            def _():
                pltpu.sync_copy(
                    zeros_hbm.at[pl.ds(16 * rpt, rem)],
                    table.at[pl.ds(16 * rpt, rem)],
                )
        plsc.subcore_barrier()

        def scat(k, buf):
            for j in range(_IPCS):
                pltpu.sync_copy(
                    rows_v.at[buf, pl.ds(j * 128, 128)],
                    table.at[idx_all.at[_IPCS * k + j]],
                    add=True,
                )

        def body(i, carry):
            k0 = 2 * i

            @pl.when(k0 < nvalid)
            def _():
                drain(0)

                @pl.when(k0 + 1 < nvalid)
                def _():
                    fire(k0 + 1, 1)
                scat(k0, 0)

                @pl.when(k0 + 1 < nvalid)
                def _():
                    drain(1)

                    @pl.when(k0 + 2 < nvalid)
                    def _():
                        fire(k0 + 2, 0)
                    scat(k0 + 1, 1)

            return carry

        lax.fori_loop(0, nk // 2, body, 0)
        plsc.subcore_barrier()
        pltpu.sync_copy(
            table.at[pl.ds(sid * rpt, rpt)], out.at[cid, pl.ds(sid * rpt, rpt)]
        )
        if rem:
            @pl.when(sid == 15)
            def _():
                pltpu.sync_copy(
                    table.at[pl.ds(16 * rpt, rem)],
                    out.at[cid, pl.ds(16 * rpt, rem)],
                )

    return sk


_BT = 8000


def _init_body(gx_ref, ea_ref, wx_ref, we_ref, inp_ref, msg_ref):
    acc = jnp.dot(gx_ref[...], wx_ref[...], preferred_element_type=jnp.float32)
    acc = acc + jnp.dot(ea_ref[...], we_ref[...], preferred_element_type=jnp.float32)
    inp_ref[...] = acc
    msg_ref[...] = jnp.maximum(acc, 0.0)


def _tc_init(gx, ea, wx, we, blk_off):
    e = gx.shape[0]
    nb = e // _BT
    return pl.pallas_call(
        _init_body,
        grid=(nb,),
        in_specs=[
            pl.BlockSpec((_BT, DIM), lambda i: (i, 0)),
            pl.BlockSpec((_BT, 16), lambda i, o=blk_off: (i + o, 0)),
            pl.BlockSpec((DIM, DIM), lambda i: (0, 0)),
            pl.BlockSpec((16, DIM), lambda i: (0, 0)),
        ],
        out_specs=[
            pl.BlockSpec((_BT, DIM), lambda i: (i, 0)),
            pl.BlockSpec((_BT, DIM), lambda i: (i, 0)),
        ],
        out_shape=[
            jax.ShapeDtypeStruct((e, DIM), jnp.float32),
            jax.ShapeDtypeStruct((e, DIM), jnp.float32),
        ],
    )(gx, ea, wx, we)


def _depth_body(msg_ref, g_ref, inp_ref, wh_ref, out_ref):
    msg = msg_ref[...]
    fwd = jnp.roll(msg, -1, axis=0)
    bwd = jnp.roll(msg, 1, axis=0)
    row = lax.broadcasted_iota(jnp.int32, msg.shape, 0)
    swapped = jnp.where((row & 1) == 0, fwd, bwd)
    t = g_ref[...].astype(jnp.float32) - swapped
    z = inp_ref[...] + jnp.dot(t, wh_ref[...], preferred_element_type=jnp.float32)
    out_ref[...] = jnp.maximum(z, 0.0)


def _tc_depth(msg, g, inp, wh_t):
    e = msg.shape[0]
    return pl.pallas_call(
        _depth_body,
        grid=(e // _BT,),
        in_specs=[
            pl.BlockSpec((_BT, DIM), lambda i: (i, 0)),
            pl.BlockSpec((_BT, DIM), lambda i: (i, 0)),
            pl.BlockSpec((_BT, DIM), lambda i: (i, 0)),
            pl.BlockSpec((DIM, DIM), lambda i: (0, 0)),
        ],
        out_specs=pl.BlockSpec((_BT, DIM), lambda i: (i, 0)),
        out_shape=jax.ShapeDtypeStruct((e, DIM), jnp.float32),
    )(msg, g, inp, wh_t)


def _combine_body(*refs):
    out_ref = refs[-1]
    acc = refs[0][0] + refs[0][1]
    for p_ref in refs[1:-1]:
        acc = acc + (p_ref[0] + p_ref[1])
    out_ref[...] = acc.astype(out_ref.dtype)


def _tc_combine(parts):
    n = parts[0].shape[1]
    bn = 1000
    return pl.pallas_call(
        _combine_body,
        grid=(n // bn,),
        in_specs=[pl.BlockSpec((2, bn, DIM), lambda i: (0, i, 0))] * len(parts),
        out_specs=pl.BlockSpec((bn, DIM), lambda i: (i, 0)),
        out_shape=jax.ShapeDtypeStruct((n, DIM), jnp.float32),
    )(*parts)


def _final_body(x_ref, *refs):
    wox_ref, wos_ref, b_ref, out_ref = refs[-4:]
    s = refs[0][0] + refs[0][1]
    for p_ref in refs[1:-4]:
        s = s + (p_ref[0] + p_ref[1])
    z = jnp.dot(x_ref[...], wox_ref[...], preferred_element_type=jnp.float32)
    z = z + jnp.dot(s, wos_ref[...], preferred_element_type=jnp.float32)
    out_ref[...] = jnp.maximum(z + b_ref[...], 0.0)


def _tc_final(x, parts, wox, wos, b2):
    n = x.shape[0]
    bn = 1000
    return pl.pallas_call(
        _final_body,
        grid=(n // bn,),
        in_specs=[pl.BlockSpec((bn, DIM), lambda i: (i, 0))]
        + [pl.BlockSpec((2, bn, DIM), lambda i: (0, i, 0))] * len(parts)
        + [
            pl.BlockSpec((DIM, DIM), lambda i: (0, 0)),
            pl.BlockSpec((DIM, DIM), lambda i: (0, 0)),
            pl.BlockSpec((1, DIM), lambda i: (0, 0)),
        ],
        out_specs=pl.BlockSpec((bn, DIM), lambda i: (i, 0)),
        out_shape=jax.ShapeDtypeStruct((n, DIM), jnp.float32),
    )(x, *parts, wox, wos, b2)


_PIECES = (1250, 1250)


def _piece_layout(r):
    assert sum(_PIECES) == r
    out = []
    off = 0
    for rows in _PIECES:
        rpw = ((rows + _NW - 1) // _NW + 7) // 8 * 8
        out.append((off, rows, rpw * _NW))
        off += rpw * _NW
    return tuple(out)


def _prep_body(ei_ref, src_ref, dst_ref, dsts_ref):
    s = ei_ref[0]
    d = ei_ref[1]
    fwd = jnp.roll(d, -1, axis=1)
    bwd = jnp.roll(d, 1, axis=1)
    lane = lax.broadcasted_iota(jnp.int32, d.shape, 1)
    ds_ = jnp.where((lane & 1) == 0, fwd, bwd)

    layout = _piece_layout(s.shape[0])

    def padded(v):
        parts = []
        row = 0
        for _, rows, rpad in layout:
            parts.append(v[row:row + rows])
            parts.append(jnp.zeros((rpad - rows, 128), jnp.int32))
            row += rows
        return jnp.concatenate(parts, axis=0)

    src_ref[...] = padded(s)
    dst_ref[...] = padded(d)
    dsts_ref[...] = padded(ds_)


def _tc_prep(ei3, rpad_total):
    r = ei3.shape[1]
    return pl.pallas_call(
        _prep_body,
        grid=(1,),
        in_specs=[pl.BlockSpec((2, r, 128), lambda i: (0, 0, 0))],
        out_specs=[pl.BlockSpec((rpad_total, 128), lambda i: (0, 0))] * 3,
        out_shape=[jax.ShapeDtypeStruct((rpad_total, 128), jnp.int32)] * 3,
    )(ei3)


def kernel(x, edge_index, edge_attr, W_i, W_h, W_o, b_o):
    n = x.shape[0]
    e = edge_attr.shape[0]
    depth = 6

    r = e // 128
    layout = _piece_layout(r)
    rpad_total = layout[-1][0] + layout[-1][2]
    np_ = len(layout)
    ei3 = edge_index.astype(jnp.int32).reshape(2, r, 128)
    src_i, dst_i, dsts_i = _tc_prep(ei3, rpad_total)
    zeros_tab = jnp.zeros((n, DIM), jnp.float32)

    wx = W_i[:, :DIM].T
    we = W_i[:, DIM:].T
    wh_t = W_h.T
    wox = W_o[:, :DIM].T
    wos = W_o[:, DIM:].T
    b2 = b_o.reshape(1, DIM)

    gathers_f = [_make_gather(n, rows * 128, off, jnp.float32) for off, rows, _ in layout]
    gathers_h = gathers_f
    scatters = [_make_scatter(n, rows * 128, off) for off, rows, _ in layout]
    blk_offs = []
    row = 0
    for _, rows, _ in layout:
        blk_offs.append(row * 128 // _BT)
        row += rows

    gx = [gathers_f[p](x, src_i) for p in range(np_)]
    inp, msg = [None] * np_, [None] * np_
    for p in range(np_):
        inp[p], msg[p] = _tc_init(gx[p], edge_attr, wx, we, blk_offs[p])
    part = [scatters[p](msg[p], dst_i, zeros_tab) for p in range(np_)]
    for _ in range(depth - 1):
        esum = _tc_combine(part)
        g = [gathers_h[p](esum, dsts_i) for p in range(np_)]
        for p in range(np_):
            msg[p] = _tc_depth(msg[p], g[p], inp[p], wh_t)
            part[p] = scatters[p](msg[p], dst_i, zeros_tab)
    return _tc_final(x, part, wox, wos, b2)
